# Initial kernel scaffold; baseline (speedup 1.0000x reference)
import jax
import jax.numpy as jnp
from jax.experimental import pallas as pl


def kernel(x, edge_index, edge_weight, W1, b1, W2, b2, Wg1, bg1, Wg2, bg2, Wl, bl, Wgen, bgen):
    raise NotImplementedError("write your pallas kernel here")


# speedup vs baseline: 16.8295x; 16.8295x over previous
import functools

import jax
import jax.numpy as jnp
from jax import lax
from jax.experimental import pallas as pl
from jax.experimental.pallas import tpu as pltpu
from jax.experimental.pallas import tpu_sc as plsc

N = 50000
E = 800000
NP = 50176
EP = 16 * 392 * 128
TCH = 392
CH = 128
RPT = NP // 16
F = 32

_mesh = lambda: plsc.VectorSubcoreMesh(
    core_axis_name="c", subcore_axis_name="s", num_cores=2, num_subcores=16)
_sc_params = lambda: pltpu.CompilerParams(use_tc_tiling_on_sc=False)

_f32 = jnp.float32
_i32 = jnp.int32


def _deg_kernel_body(dst_hbm, ew_hbm, deg_hbm, dst_v, ew_v, nbuf, degsh):
    cid = lax.axis_index("c")
    sid = lax.axis_index("s")

    def _fill(i, _):
        nbuf[pl.ds(i * 16, 16)] = jnp.full((16,), 1.0, _f32)
        return 0
    lax.fori_loop(0, RPT // 16, _fill, 0)
    pltpu.sync_copy(nbuf, degsh.at[pl.ds(sid * RPT, RPT)])
    plsc.subcore_barrier()

    pltpu.sync_copy(dst_hbm.at[sid], dst_v)
    pltpu.sync_copy(ew_hbm.at[sid], ew_v)

    def _acc(j, _):
        pltpu.sync_copy(ew_v.at[pl.ds(j * CH, CH)], degsh.at[dst_v.at[j]],
                        add=True)
        return 0
    lax.fori_loop(0, TCH, _acc, 0)
    plsc.subcore_barrier()

    half = RPT // 2
    base = cid * (NP // 2) + sid * half
    dv = nbuf.at[pl.ds(0, half)]
    pltpu.sync_copy(degsh.at[pl.ds(base, half)], dv)
    pltpu.sync_copy(dv, deg_hbm.at[pl.ds(base, half)])


def _compute_deg(dstp, ewp):
    k = pl.kernel(
        _deg_kernel_body,
        out_type=jax.ShapeDtypeStruct((NP,), _f32),
        mesh=_mesh(),
        scratch_types=[
            pltpu.VMEM((TCH, CH), _i32),
            pltpu.VMEM((TCH * CH,), _f32),
            pltpu.VMEM((RPT,), _f32),
            pltpu.VMEM_SHARED((NP,), _f32),
        ],
        compiler_params=_sc_params(),
    )
    return k(dstp, ewp)


GC = 28
NG = TCH // GC


def _agg_kernel_body(ha_hbm, hb_hbm, src_hbm, dst_hbm, ew_hbm, out_hbm,
                     src_g, dst_g, ew_g, buf, accum):
    cid = lax.axis_index("c")
    sid = lax.axis_index("s")

    def _zb(r, _):
        buf[r, pl.ds(0, 16)] = jnp.zeros((16,), _f32)
        buf[r, pl.ds(16, 16)] = jnp.zeros((16,), _f32)
        return 0
    lax.fori_loop(0, CH, _zb, 0)
    zbase = sid * RPT
    for t in range(RPT // CH):
        pltpu.sync_copy(buf, accum.at[pl.ds(zbase + t * CH, CH)])
    rem = RPT - (RPT // CH) * CH
    if rem:
        pltpu.sync_copy(buf.at[pl.ds(0, rem)],
                        accum.at[pl.ds(zbase + (RPT // CH) * CH, rem)])
    plsc.subcore_barrier()

    def _main(h_hbm):
        def _grp(grp, _):
            pltpu.sync_copy(src_hbm.at[sid].at[pl.ds(grp * GC, GC)], src_g)
            pltpu.sync_copy(dst_hbm.at[sid].at[pl.ds(grp * GC, GC)], dst_g)
            pltpu.sync_copy(ew_hbm.at[sid].at[pl.ds(grp * GC * CH, GC * CH)],
                            ew_g)

            def _row(j, _):
                pltpu.sync_copy(h_hbm.at[src_g.at[j]], buf)

                def _scale(g, _):
                    w16 = ew_g[pl.ds(j * CH + g * 16, 16)]
                    for t in range(16):
                        e = g * 16 + t
                        wv = w16[jnp.full((16,), t, _i32)]
                        buf[e, pl.ds(0, 16)] = buf[e, pl.ds(0, 16)] * wv
                        buf[e, pl.ds(16, 16)] = buf[e, pl.ds(16, 16)] * wv
                    return 0
                lax.fori_loop(0, CH // 16, _scale, 0)
                pltpu.sync_copy(buf, accum.at[dst_g.at[j]], add=True)
                return 0
            lax.fori_loop(0, GC, _row, 0)
            return 0
        lax.fori_loop(0, NG, _grp, 0)

    @pl.when(cid == 0)
    def _():
        _main(ha_hbm)

    @pl.when(cid == 1)
    def _():
        _main(hb_hbm)

    plsc.subcore_barrier()
    for t in range(RPT // CH):
        r = sid * RPT + t * CH
        pltpu.sync_copy(accum.at[pl.ds(r, CH)], buf)
        pltpu.sync_copy(buf, out_hbm.at[cid].at[pl.ds(r, CH)])
    if RPT % CH:
        r = sid * RPT + (RPT // CH) * CH
        pltpu.sync_copy(accum.at[pl.ds(r, RPT % CH)],
                        buf.at[pl.ds(0, RPT % CH)])
        pltpu.sync_copy(buf.at[pl.ds(0, RPT % CH)],
                        out_hbm.at[cid].at[pl.ds(r, RPT % CH)])


def _aggregate(ha, hb, srcp, dstp, ewp):
    k = pl.kernel(
        _agg_kernel_body,
        out_type=jax.ShapeDtypeStruct((2, NP, F), _f32),
        mesh=_mesh(),
        scratch_types=[
            pltpu.VMEM((GC, CH), _i32),
            pltpu.VMEM((GC, CH), _i32),
            pltpu.VMEM((GC * CH,), _f32),
            pltpu.VMEM((CH, F), _f32),
            pltpu.VMEM_SHARED((NP, F), _f32),
        ],
        compiler_params=_sc_params(),
    )
    return k(ha, hb, srcp, dstp, ewp)


_GRID = NP // RPT


def _gelu(t):
    return t * 0.5 * (1.0 + lax.erf(t * 0.7071067811865476))


def _dinv(deg):
    return jnp.where(deg > 0, lax.rsqrt(deg), 0.0)


def _mlp_body(x_ref, deg_ref, w1_ref, b1_ref, w2_ref, b2_ref, wg1_ref,
              ha_ref, hb_ref):
    z = _gelu(jnp.dot(x_ref[...], w1_ref[...],
                      preferred_element_type=_f32) + b1_ref[...])
    z = _gelu(jnp.dot(z, w2_ref[...],
                      preferred_element_type=_f32) + b2_ref[...])
    h = jnp.dot(z, wg1_ref[...], preferred_element_type=_f32)
    hp = h * _dinv(deg_ref[...])
    ha_ref[...] = hp[:, :F]
    hb_ref[...] = hp[:, F:]


def _mlp_stage(xp, deg, W1, b1, W2, b2, Wg1):
    row = lambda i: (i, 0)
    return pl.pallas_call(
        _mlp_body,
        grid=(_GRID,),
        in_specs=[
            pl.BlockSpec((RPT, 128), row),
            pl.BlockSpec((RPT, 1), row),
            pl.BlockSpec((128, 256), lambda i: (0, 0)),
            pl.BlockSpec((1, 256), lambda i: (0, 0)),
            pl.BlockSpec((256, 256), lambda i: (0, 0)),
            pl.BlockSpec((1, 256), lambda i: (0, 0)),
            pl.BlockSpec((256, 64), lambda i: (0, 0)),
        ],
        out_specs=[pl.BlockSpec((RPT, F), row), pl.BlockSpec((RPT, F), row)],
        out_shape=[jax.ShapeDtypeStruct((NP, F), _f32)] * 2,
    )(xp, deg, W1, b1, W2, b2, Wg1)


def _gcn_mid_body(agg_ref, ha_ref, hb_ref, deg_ref, bg_ref, wg_ref,
                  oa_ref, ob_ref):
    dinv = _dinv(deg_ref[...])
    agg = jnp.concatenate([agg_ref[0], agg_ref[1]], axis=1)
    hp = jnp.concatenate([ha_ref[...], hb_ref[...]], axis=1)
    z = _gelu((agg + hp) * dinv + bg_ref[...])
    h2 = jnp.dot(z, wg_ref[...], preferred_element_type=_f32) * dinv
    oa_ref[...] = h2[:, :F]
    ob_ref[...] = h2[:, F:]


def _gcn_mid_stage(agg, ha, hb, deg, bg1, Wg2):
    row = lambda i: (i, 0)
    return pl.pallas_call(
        _gcn_mid_body,
        grid=(_GRID,),
        in_specs=[
            pl.BlockSpec((2, RPT, F), lambda i: (0, i, 0)),
            pl.BlockSpec((RPT, F), row),
            pl.BlockSpec((RPT, F), row),
            pl.BlockSpec((RPT, 1), row),
            pl.BlockSpec((1, 64), lambda i: (0, 0)),
            pl.BlockSpec((64, 64), lambda i: (0, 0)),
        ],
        out_specs=[pl.BlockSpec((RPT, F), row), pl.BlockSpec((RPT, F), row)],
        out_shape=[jax.ShapeDtypeStruct((NP, F), _f32)] * 2,
    )(agg, ha, hb, deg, bg1, Wg2)


def _final_body(agg_ref, ha_ref, hb_ref, deg_ref, bg_ref, wl_ref, bl_ref,
                wgen_ref, bgen_ref, out_ref):
    agg = jnp.concatenate([agg_ref[0], agg_ref[1]], axis=1)
    hp = jnp.concatenate([ha_ref[...], hb_ref[...]], axis=1)
    z = _gelu((agg + hp) * _dinv(deg_ref[...]) + bg_ref[...])
    t = jnp.dot(z, wl_ref[...], preferred_element_type=_f32) + bl_ref[...]
    out_ref[...] = (jnp.dot(t, wgen_ref[...], preferred_element_type=_f32)
                    + bgen_ref[...])


def _final_stage(agg, ha, hb, deg, bg2, Wl, bl, Wgen, bgen):
    row = lambda i: (i, 0)
    return pl.pallas_call(
        _final_body,
        grid=(_GRID,),
        in_specs=[
            pl.BlockSpec((2, RPT, F), lambda i: (0, i, 0)),
            pl.BlockSpec((RPT, F), row),
            pl.BlockSpec((RPT, F), row),
            pl.BlockSpec((RPT, 1), row),
            pl.BlockSpec((1, 64), lambda i: (0, 0)),
            pl.BlockSpec((64, 64), lambda i: (0, 0)),
            pl.BlockSpec((1, 64), lambda i: (0, 0)),
            pl.BlockSpec((64, 64), lambda i: (0, 0)),
            pl.BlockSpec((1, 64), lambda i: (0, 0)),
        ],
        out_specs=pl.BlockSpec((RPT, 64), row),
        out_shape=jax.ShapeDtypeStruct((NP, 64), _f32),
    )(agg, ha, hb, deg, bg2, Wl, bl, Wgen, bgen)


def kernel(x, edge_index, edge_weight, W1, b1, W2, b2, Wg1, bg1, Wg2, bg2,
           Wl, bl, Wgen, bgen):
    src = edge_index[0]
    dst = edge_index[1]
    pad_i = jnp.zeros((EP - E,), _i32)
    srcp = jnp.concatenate([src, pad_i]).reshape(16, TCH, CH)
    dstp = jnp.concatenate([dst, pad_i]).reshape(16, TCH, CH)
    ewp = jnp.concatenate([edge_weight,
                           jnp.zeros((EP - E,), _f32)]).reshape(16, TCH * CH)
    xp = jnp.concatenate([x, jnp.zeros((NP - N, 128), _f32)])

    deg = _compute_deg(dstp, ewp).reshape(NP, 1)

    ha, hb = _mlp_stage(xp, deg, W1, b1.reshape(1, -1), W2,
                        b2.reshape(1, -1), Wg1)
    agg1 = _aggregate(ha, hb, srcp, dstp, ewp)
    h2a, h2b = _gcn_mid_stage(agg1, ha, hb, deg, bg1.reshape(1, -1), Wg2)
    agg2 = _aggregate(h2a, h2b, srcp, dstp, ewp)
    out = _final_stage(agg2, h2a, h2b, deg, bg2.reshape(1, -1), Wl,
                       bl.reshape(1, -1), Wgen, bgen.reshape(1, -1))
    return out[:N]


# speedup vs baseline: 21.0662x; 1.2517x over previous
import functools

import jax
import jax.numpy as jnp
from jax import lax
from jax.experimental import pallas as pl
from jax.experimental.pallas import tpu as pltpu
from jax.experimental.pallas import tpu_sc as plsc

N = 50000
E = 800000
NP = 50176
EP = 16 * 392 * 128
TCH = 392
CH = 128
RPT = NP // 16
F = 32

_mesh = lambda: plsc.VectorSubcoreMesh(
    core_axis_name="c", subcore_axis_name="s", num_cores=2, num_subcores=16)
_sc_params = lambda: pltpu.CompilerParams(use_tc_tiling_on_sc=False)

_f32 = jnp.float32
_i32 = jnp.int32


def _deg_kernel_body(dst_hbm, ew_hbm, deg_hbm, dst_v, ew_v, nbuf, degsh):
    cid = lax.axis_index("c")
    sid = lax.axis_index("s")

    def _fill(i, _):
        nbuf[pl.ds(i * 16, 16)] = jnp.full((16,), 1.0, _f32)
        return 0
    lax.fori_loop(0, RPT // 16, _fill, 0)
    pltpu.sync_copy(nbuf, degsh.at[pl.ds(sid * RPT, RPT)])
    plsc.subcore_barrier()

    pltpu.sync_copy(dst_hbm.at[sid], dst_v)
    pltpu.sync_copy(ew_hbm.at[sid], ew_v)

    def _acc(j, _):
        pltpu.sync_copy(ew_v.at[pl.ds(j * CH, CH)], degsh.at[dst_v.at[j]],
                        add=True)
        return 0
    lax.fori_loop(0, TCH, _acc, 0)
    plsc.subcore_barrier()

    half = RPT // 2
    base = cid * (NP // 2) + sid * half
    dv = nbuf.at[pl.ds(0, half)]
    pltpu.sync_copy(degsh.at[pl.ds(base, half)], dv)
    pltpu.sync_copy(dv, deg_hbm.at[pl.ds(base, half)])


def _compute_deg(dstp, ewp):
    k = pl.kernel(
        _deg_kernel_body,
        out_type=jax.ShapeDtypeStruct((NP,), _f32),
        mesh=_mesh(),
        scratch_types=[
            pltpu.VMEM((TCH, CH), _i32),
            pltpu.VMEM((TCH * CH,), _f32),
            pltpu.VMEM((RPT,), _f32),
            pltpu.VMEM_SHARED((NP,), _f32),
        ],
        compiler_params=_sc_params(),
    )
    return k(dstp, ewp)


GC = 28
NG = TCH // GC


def _agg_kernel_body(ha_hbm, hb_hbm, src_hbm, dst_hbm, ew_hbm, out_hbm,
                     src_g, dst_g, ew_g, bufs, accum, sem0, sem1):
    cid = lax.axis_index("c")
    sid = lax.axis_index("s")
    buf = bufs.at[0]

    def _zb(r, _):
        buf[r, pl.ds(0, 16)] = jnp.zeros((16,), _f32)
        buf[r, pl.ds(16, 16)] = jnp.zeros((16,), _f32)
        return 0
    lax.fori_loop(0, CH, _zb, 0)
    zbase = sid * RPT
    for t in range(RPT // CH):
        pltpu.sync_copy(buf, accum.at[pl.ds(zbase + t * CH, CH)])
    rem = RPT - (RPT // CH) * CH
    if rem:
        pltpu.sync_copy(buf.at[pl.ds(0, rem)],
                        accum.at[pl.ds(zbase + (RPT // CH) * CH, rem)])
    plsc.subcore_barrier()

    sems = (sem0, sem1)

    def _main(h_hbm):
        def _start(j, b):
            pltpu.async_copy(h_hbm.at[src_g.at[j]], bufs.at[b], sems[b])

        def _wait(j, b):
            pltpu.make_async_copy(h_hbm.at[src_g.at[j]], bufs.at[b],
                                  sems[b]).wait()

        def _consume(j, b):
            bb = bufs.at[b]

            def _scale(g, _):
                w16 = ew_g[pl.ds(j * CH + g * 16, 16)]
                for t in range(16):
                    e = g * 16 + t
                    wv = w16[jnp.full((16,), t, _i32)]
                    bb[e, pl.ds(0, 16)] = bb[e, pl.ds(0, 16)] * wv
                    bb[e, pl.ds(16, 16)] = bb[e, pl.ds(16, 16)] * wv
                return 0
            lax.fori_loop(0, CH // 16, _scale, 0)
            pltpu.sync_copy(bb, accum.at[dst_g.at[j]], add=True)

        def _grp(grp, _):
            pltpu.sync_copy(src_hbm.at[sid].at[pl.ds(grp * GC, GC)], src_g)
            pltpu.sync_copy(dst_hbm.at[sid].at[pl.ds(grp * GC, GC)], dst_g)
            pltpu.sync_copy(ew_hbm.at[sid].at[pl.ds(grp * GC * CH, GC * CH)],
                            ew_g)

            _start(0, 0)

            def _row2(j2, _):
                j = j2 * 2
                _wait(j, 0)
                _start(j + 1, 1)
                _consume(j, 0)
                _wait(j + 1, 1)

                @pl.when(j + 2 < GC)
                def _():
                    _start(j + 2, 0)
                _consume(j + 1, 1)
                return 0
            lax.fori_loop(0, GC // 2, _row2, 0)
            return 0
        lax.fori_loop(0, NG, _grp, 0)

    @pl.when(cid == 0)
    def _():
        _main(ha_hbm)

    @pl.when(cid == 1)
    def _():
        _main(hb_hbm)

    plsc.subcore_barrier()
    for t in range(RPT // CH):
        r = sid * RPT + t * CH
        pltpu.sync_copy(accum.at[pl.ds(r, CH)], buf)
        pltpu.sync_copy(buf, out_hbm.at[cid].at[pl.ds(r, CH)])
    if RPT % CH:
        r = sid * RPT + (RPT // CH) * CH
        pltpu.sync_copy(accum.at[pl.ds(r, RPT % CH)],
                        buf.at[pl.ds(0, RPT % CH)])
        pltpu.sync_copy(buf.at[pl.ds(0, RPT % CH)],
                        out_hbm.at[cid].at[pl.ds(r, RPT % CH)])


def _aggregate(ha, hb, srcp, dstp, ewp):
    k = pl.kernel(
        _agg_kernel_body,
        out_type=jax.ShapeDtypeStruct((2, NP, F), _f32),
        mesh=_mesh(),
        scratch_types=[
            pltpu.VMEM((GC, CH), _i32),
            pltpu.VMEM((GC, CH), _i32),
            pltpu.VMEM((GC * CH,), _f32),
            pltpu.VMEM((2, CH, F), _f32),
            pltpu.VMEM_SHARED((NP, F), _f32),
            pltpu.SemaphoreType.DMA,
            pltpu.SemaphoreType.DMA,
        ],
        compiler_params=_sc_params(),
    )
    return k(ha, hb, srcp, dstp, ewp)


_GRID = NP // RPT


def _gelu(t):
    return t * 0.5 * (1.0 + lax.erf(t * 0.7071067811865476))


def _dinv(deg):
    return jnp.where(deg > 0, lax.rsqrt(deg), 0.0)


def _mlp_body(x_ref, deg_ref, w1_ref, b1_ref, w2_ref, b2_ref, wg1_ref,
              ha_ref, hb_ref):
    z = _gelu(jnp.dot(x_ref[...], w1_ref[...],
                      preferred_element_type=_f32) + b1_ref[...])
    z = _gelu(jnp.dot(z, w2_ref[...],
                      preferred_element_type=_f32) + b2_ref[...])
    h = jnp.dot(z, wg1_ref[...], preferred_element_type=_f32)
    hp = h * _dinv(deg_ref[...])
    ha_ref[...] = hp[:, :F]
    hb_ref[...] = hp[:, F:]


def _mlp_stage(xp, deg, W1, b1, W2, b2, Wg1):
    row = lambda i: (i, 0)
    return pl.pallas_call(
        _mlp_body,
        grid=(_GRID,),
        in_specs=[
            pl.BlockSpec((RPT, 128), row),
            pl.BlockSpec((RPT, 1), row),
            pl.BlockSpec((128, 256), lambda i: (0, 0)),
            pl.BlockSpec((1, 256), lambda i: (0, 0)),
            pl.BlockSpec((256, 256), lambda i: (0, 0)),
            pl.BlockSpec((1, 256), lambda i: (0, 0)),
            pl.BlockSpec((256, 64), lambda i: (0, 0)),
        ],
        out_specs=[pl.BlockSpec((RPT, F), row), pl.BlockSpec((RPT, F), row)],
        out_shape=[jax.ShapeDtypeStruct((NP, F), _f32)] * 2,
    )(xp, deg, W1, b1, W2, b2, Wg1)


def _gcn_mid_body(agg_ref, ha_ref, hb_ref, deg_ref, bg_ref, wg_ref,
                  oa_ref, ob_ref):
    dinv = _dinv(deg_ref[...])
    agg = jnp.concatenate([agg_ref[0], agg_ref[1]], axis=1)
    hp = jnp.concatenate([ha_ref[...], hb_ref[...]], axis=1)
    z = _gelu((agg + hp) * dinv + bg_ref[...])
    h2 = jnp.dot(z, wg_ref[...], preferred_element_type=_f32) * dinv
    oa_ref[...] = h2[:, :F]
    ob_ref[...] = h2[:, F:]


def _gcn_mid_stage(agg, ha, hb, deg, bg1, Wg2):
    row = lambda i: (i, 0)
    return pl.pallas_call(
        _gcn_mid_body,
        grid=(_GRID,),
        in_specs=[
            pl.BlockSpec((2, RPT, F), lambda i: (0, i, 0)),
            pl.BlockSpec((RPT, F), row),
            pl.BlockSpec((RPT, F), row),
            pl.BlockSpec((RPT, 1), row),
            pl.BlockSpec((1, 64), lambda i: (0, 0)),
            pl.BlockSpec((64, 64), lambda i: (0, 0)),
        ],
        out_specs=[pl.BlockSpec((RPT, F), row), pl.BlockSpec((RPT, F), row)],
        out_shape=[jax.ShapeDtypeStruct((NP, F), _f32)] * 2,
    )(agg, ha, hb, deg, bg1, Wg2)


def _final_body(agg_ref, ha_ref, hb_ref, deg_ref, bg_ref, wl_ref, bl_ref,
                wgen_ref, bgen_ref, out_ref):
    agg = jnp.concatenate([agg_ref[0], agg_ref[1]], axis=1)
    hp = jnp.concatenate([ha_ref[...], hb_ref[...]], axis=1)
    z = _gelu((agg + hp) * _dinv(deg_ref[...]) + bg_ref[...])
    t = jnp.dot(z, wl_ref[...], preferred_element_type=_f32) + bl_ref[...]
    out_ref[...] = (jnp.dot(t, wgen_ref[...], preferred_element_type=_f32)
                    + bgen_ref[...])


def _final_stage(agg, ha, hb, deg, bg2, Wl, bl, Wgen, bgen):
    row = lambda i: (i, 0)
    return pl.pallas_call(
        _final_body,
        grid=(_GRID,),
        in_specs=[
            pl.BlockSpec((2, RPT, F), lambda i: (0, i, 0)),
            pl.BlockSpec((RPT, F), row),
            pl.BlockSpec((RPT, F), row),
            pl.BlockSpec((RPT, 1), row),
            pl.BlockSpec((1, 64), lambda i: (0, 0)),
            pl.BlockSpec((64, 64), lambda i: (0, 0)),
            pl.BlockSpec((1, 64), lambda i: (0, 0)),
            pl.BlockSpec((64, 64), lambda i: (0, 0)),
            pl.BlockSpec((1, 64), lambda i: (0, 0)),
        ],
        out_specs=pl.BlockSpec((RPT, 64), row),
        out_shape=jax.ShapeDtypeStruct((NP, 64), _f32),
    )(agg, ha, hb, deg, bg2, Wl, bl, Wgen, bgen)


def kernel(x, edge_index, edge_weight, W1, b1, W2, b2, Wg1, bg1, Wg2, bg2,
           Wl, bl, Wgen, bgen):
    src = edge_index[0]
    dst = edge_index[1]
    pad_i = jnp.zeros((EP - E,), _i32)
    srcp = jnp.concatenate([src, pad_i]).reshape(16, TCH, CH)
    dstp = jnp.concatenate([dst, pad_i]).reshape(16, TCH, CH)
    ewp = jnp.concatenate([edge_weight,
                           jnp.zeros((EP - E,), _f32)]).reshape(16, TCH * CH)
    xp = jnp.concatenate([x, jnp.zeros((NP - N, 128), _f32)])

    deg = _compute_deg(dstp, ewp).reshape(NP, 1)

    ha, hb = _mlp_stage(xp, deg, W1, b1.reshape(1, -1), W2,
                        b2.reshape(1, -1), Wg1)
    agg1 = _aggregate(ha, hb, srcp, dstp, ewp)
    h2a, h2b = _gcn_mid_stage(agg1, ha, hb, deg, bg1.reshape(1, -1), Wg2)
    agg2 = _aggregate(h2a, h2b, srcp, dstp, ewp)
    out = _final_stage(agg2, h2a, h2b, deg, bg2.reshape(1, -1), Wl,
                       bl.reshape(1, -1), Wgen, bgen.reshape(1, -1))
    return out[:N]


# speedup vs baseline: 25.2971x; 1.2008x over previous
import functools

import jax
import jax.numpy as jnp
from jax import lax
from jax.experimental import pallas as pl
from jax.experimental.pallas import tpu as pltpu
from jax.experimental.pallas import tpu_sc as plsc

N = 50000
E = 800000
NP = 50176
EP = 16 * 392 * 128
TCH = 392
CH = 128
RPT = NP // 16
F = 32

_mesh = lambda: plsc.VectorSubcoreMesh(
    core_axis_name="c", subcore_axis_name="s", num_cores=2, num_subcores=16)
_sc_params = lambda: pltpu.CompilerParams(use_tc_tiling_on_sc=False)

_f32 = jnp.float32
_i32 = jnp.int32


def _deg_kernel_body(dst_hbm, ew_hbm, deg_hbm, dst_v, ew_v, nbuf, degsh):
    cid = lax.axis_index("c")
    sid = lax.axis_index("s")

    def _fill(i, _):
        nbuf[pl.ds(i * 16, 16)] = jnp.full((16,), 1.0, _f32)
        return 0
    lax.fori_loop(0, RPT // 16, _fill, 0)
    pltpu.sync_copy(nbuf, degsh.at[pl.ds(sid * RPT, RPT)])
    plsc.subcore_barrier()

    pltpu.sync_copy(dst_hbm.at[sid], dst_v)
    pltpu.sync_copy(ew_hbm.at[sid], ew_v)

    def _acc(j, _):
        pltpu.sync_copy(ew_v.at[pl.ds(j * CH, CH)], degsh.at[dst_v.at[j]],
                        add=True)
        return 0
    lax.fori_loop(0, TCH, _acc, 0)
    plsc.subcore_barrier()

    half = RPT // 2
    base = cid * (NP // 2) + sid * half
    dv = nbuf.at[pl.ds(0, half)]
    pltpu.sync_copy(degsh.at[pl.ds(base, half)], dv)
    pltpu.sync_copy(dv, deg_hbm.at[pl.ds(base, half)])


def _compute_deg(dstp, ewp):
    k = pl.kernel(
        _deg_kernel_body,
        out_type=jax.ShapeDtypeStruct((NP,), _f32),
        mesh=_mesh(),
        scratch_types=[
            pltpu.VMEM((TCH, CH), _i32),
            pltpu.VMEM((TCH * CH,), _f32),
            pltpu.VMEM((RPT,), _f32),
            pltpu.VMEM_SHARED((NP,), _f32),
        ],
        compiler_params=_sc_params(),
    )
    return k(dstp, ewp)


GC = 28
NG = TCH // GC


def _agg_kernel_body(ha_hbm, hb_hbm, src_hbm, dst_hbm, ew_hbm, out_hbm,
                     src_g, dst_g, ew_g, bufs, accum, *sems):
    cid = lax.axis_index("c")
    sid = lax.axis_index("s")
    gsems, ssems = sems[:4], sems[4:]
    buf = bufs.at[0]

    def _zb(r, _):
        buf[r, pl.ds(0, 16)] = jnp.zeros((16,), _f32)
        buf[r, pl.ds(16, 16)] = jnp.zeros((16,), _f32)
        return 0
    lax.fori_loop(0, CH, _zb, 0)
    zbase = sid * RPT
    for t in range(RPT // CH):
        pltpu.sync_copy(buf, accum.at[pl.ds(zbase + t * CH, CH)])
    rem = RPT - (RPT // CH) * CH
    if rem:
        pltpu.sync_copy(buf.at[pl.ds(0, rem)],
                        accum.at[pl.ds(zbase + (RPT // CH) * CH, rem)])
    plsc.subcore_barrier()

    def _main(h_hbm):
        def _start_g(j, b):
            pltpu.async_copy(h_hbm.at[src_g.at[j]], bufs.at[b], gsems[b])

        def _wait_g(j, b):
            pltpu.make_async_copy(h_hbm.at[src_g.at[j]], bufs.at[b],
                                  gsems[b]).wait()

        def _start_s(j, b):
            pltpu.async_copy(bufs.at[b], accum.at[dst_g.at[j]], ssems[b],
                             add=True)

        def _wait_s(j, b):
            pltpu.make_async_copy(bufs.at[b], accum.at[dst_g.at[j]],
                                  ssems[b]).wait()

        def _scale(j, b):
            bb = bufs.at[b]

            def _sc(g, _):
                w16 = ew_g[pl.ds(j * CH + g * 16, 16)]
                for t in range(16):
                    e = g * 16 + t
                    wv = w16[jnp.full((16,), t, _i32)]
                    bb[e, pl.ds(0, 16)] = bb[e, pl.ds(0, 16)] * wv
                    bb[e, pl.ds(16, 16)] = bb[e, pl.ds(16, 16)] * wv
                return 0
            lax.fori_loop(0, CH // 16, _sc, 0)

        def _row(j, t, do_wait_s, do_start_g):
            _wait_g(j, t)
            _scale(j, t)
            _start_s(j, t)
            if do_wait_s:
                _wait_s(j - 2, (t - 2) % 4)
            if do_start_g:
                _start_g(j + 2, (t + 2) % 4)

        def _grp(grp, _):
            pltpu.sync_copy(src_hbm.at[sid].at[pl.ds(grp * GC, GC)], src_g)
            pltpu.sync_copy(dst_hbm.at[sid].at[pl.ds(grp * GC, GC)], dst_g)
            pltpu.sync_copy(ew_hbm.at[sid].at[pl.ds(grp * GC * CH, GC * CH)],
                            ew_g)

            _start_g(0, 0)
            _start_g(1, 1)
            _row(0, 0, False, True)
            _row(1, 1, False, True)

            def _q(q, _):
                j0 = 2 + q * 4
                for i in range(4):
                    _row(j0 + i, (2 + i) % 4, True, True)
                return 0
            lax.fori_loop(0, (GC - 4) // 4, _q, 0)
            _row(GC - 2, (GC - 2) % 4, True, False)
            _row(GC - 1, (GC - 1) % 4, True, False)
            _wait_s(GC - 2, (GC - 2) % 4)
            _wait_s(GC - 1, (GC - 1) % 4)
            return 0
        lax.fori_loop(0, NG, _grp, 0)

    @pl.when(cid == 0)
    def _():
        _main(ha_hbm)

    @pl.when(cid == 1)
    def _():
        _main(hb_hbm)

    plsc.subcore_barrier()
    for t in range(RPT // CH):
        r = sid * RPT + t * CH
        pltpu.sync_copy(accum.at[pl.ds(r, CH)], buf)
        pltpu.sync_copy(buf, out_hbm.at[cid].at[pl.ds(r, CH)])
    if RPT % CH:
        r = sid * RPT + (RPT // CH) * CH
        pltpu.sync_copy(accum.at[pl.ds(r, RPT % CH)],
                        buf.at[pl.ds(0, RPT % CH)])
        pltpu.sync_copy(buf.at[pl.ds(0, RPT % CH)],
                        out_hbm.at[cid].at[pl.ds(r, RPT % CH)])


def _aggregate(ha, hb, srcp, dstp, ewp):
    k = pl.kernel(
        _agg_kernel_body,
        out_type=jax.ShapeDtypeStruct((2, NP, F), _f32),
        mesh=_mesh(),
        scratch_types=[
            pltpu.VMEM((GC, CH), _i32),
            pltpu.VMEM((GC, CH), _i32),
            pltpu.VMEM((GC * CH,), _f32),
            pltpu.VMEM((4, CH, F), _f32),
            pltpu.VMEM_SHARED((NP, F), _f32),
        ] + [pltpu.SemaphoreType.DMA] * 8,
        compiler_params=_sc_params(),
    )
    return k(ha, hb, srcp, dstp, ewp)


_GRID = NP // RPT


def _gelu(t):
    return t * 0.5 * (1.0 + lax.erf(t * 0.7071067811865476))


def _dinv(deg):
    return jnp.where(deg > 0, lax.rsqrt(deg), 0.0)


def _mlp_body(x_ref, deg_ref, w1_ref, b1_ref, w2_ref, b2_ref, wg1_ref,
              ha_ref, hb_ref):
    z = _gelu(jnp.dot(x_ref[...], w1_ref[...],
                      preferred_element_type=_f32) + b1_ref[...])
    z = _gelu(jnp.dot(z, w2_ref[...],
                      preferred_element_type=_f32) + b2_ref[...])
    h = jnp.dot(z, wg1_ref[...], preferred_element_type=_f32)
    hp = h * _dinv(deg_ref[...])
    ha_ref[...] = hp[:, :F]
    hb_ref[...] = hp[:, F:]


def _mlp_stage(xp, deg, W1, b1, W2, b2, Wg1):
    row = lambda i: (i, 0)
    return pl.pallas_call(
        _mlp_body,
        grid=(_GRID,),
        in_specs=[
            pl.BlockSpec((RPT, 128), row),
            pl.BlockSpec((RPT, 1), row),
            pl.BlockSpec((128, 256), lambda i: (0, 0)),
            pl.BlockSpec((1, 256), lambda i: (0, 0)),
            pl.BlockSpec((256, 256), lambda i: (0, 0)),
            pl.BlockSpec((1, 256), lambda i: (0, 0)),
            pl.BlockSpec((256, 64), lambda i: (0, 0)),
        ],
        out_specs=[pl.BlockSpec((RPT, F), row), pl.BlockSpec((RPT, F), row)],
        out_shape=[jax.ShapeDtypeStruct((NP, F), _f32)] * 2,
    )(xp, deg, W1, b1, W2, b2, Wg1)


def _gcn_mid_body(agg_ref, ha_ref, hb_ref, deg_ref, bg_ref, wg_ref,
                  oa_ref, ob_ref):
    dinv = _dinv(deg_ref[...])
    agg = jnp.concatenate([agg_ref[0], agg_ref[1]], axis=1)
    hp = jnp.concatenate([ha_ref[...], hb_ref[...]], axis=1)
    z = _gelu((agg + hp) * dinv + bg_ref[...])
    h2 = jnp.dot(z, wg_ref[...], preferred_element_type=_f32) * dinv
    oa_ref[...] = h2[:, :F]
    ob_ref[...] = h2[:, F:]


def _gcn_mid_stage(agg, ha, hb, deg, bg1, Wg2):
    row = lambda i: (i, 0)
    return pl.pallas_call(
        _gcn_mid_body,
        grid=(_GRID,),
        in_specs=[
            pl.BlockSpec((2, RPT, F), lambda i: (0, i, 0)),
            pl.BlockSpec((RPT, F), row),
            pl.BlockSpec((RPT, F), row),
            pl.BlockSpec((RPT, 1), row),
            pl.BlockSpec((1, 64), lambda i: (0, 0)),
            pl.BlockSpec((64, 64), lambda i: (0, 0)),
        ],
        out_specs=[pl.BlockSpec((RPT, F), row), pl.BlockSpec((RPT, F), row)],
        out_shape=[jax.ShapeDtypeStruct((NP, F), _f32)] * 2,
    )(agg, ha, hb, deg, bg1, Wg2)


def _final_body(agg_ref, ha_ref, hb_ref, deg_ref, bg_ref, wl_ref, bl_ref,
                wgen_ref, bgen_ref, out_ref):
    agg = jnp.concatenate([agg_ref[0], agg_ref[1]], axis=1)
    hp = jnp.concatenate([ha_ref[...], hb_ref[...]], axis=1)
    z = _gelu((agg + hp) * _dinv(deg_ref[...]) + bg_ref[...])
    t = jnp.dot(z, wl_ref[...], preferred_element_type=_f32) + bl_ref[...]
    out_ref[...] = (jnp.dot(t, wgen_ref[...], preferred_element_type=_f32)
                    + bgen_ref[...])


def _final_stage(agg, ha, hb, deg, bg2, Wl, bl, Wgen, bgen):
    row = lambda i: (i, 0)
    return pl.pallas_call(
        _final_body,
        grid=(_GRID,),
        in_specs=[
            pl.BlockSpec((2, RPT, F), lambda i: (0, i, 0)),
            pl.BlockSpec((RPT, F), row),
            pl.BlockSpec((RPT, F), row),
            pl.BlockSpec((RPT, 1), row),
            pl.BlockSpec((1, 64), lambda i: (0, 0)),
            pl.BlockSpec((64, 64), lambda i: (0, 0)),
            pl.BlockSpec((1, 64), lambda i: (0, 0)),
            pl.BlockSpec((64, 64), lambda i: (0, 0)),
            pl.BlockSpec((1, 64), lambda i: (0, 0)),
        ],
        out_specs=pl.BlockSpec((RPT, 64), row),
        out_shape=jax.ShapeDtypeStruct((NP, 64), _f32),
    )(agg, ha, hb, deg, bg2, Wl, bl, Wgen, bgen)


def kernel(x, edge_index, edge_weight, W1, b1, W2, b2, Wg1, bg1, Wg2, bg2,
           Wl, bl, Wgen, bgen):
    src = edge_index[0]
    dst = edge_index[1]
    pad_i = jnp.zeros((EP - E,), _i32)
    srcp = jnp.concatenate([src, pad_i]).reshape(16, TCH, CH)
    dstp = jnp.concatenate([dst, pad_i]).reshape(16, TCH, CH)
    ewp = jnp.concatenate([edge_weight,
                           jnp.zeros((EP - E,), _f32)]).reshape(16, TCH * CH)
    xp = jnp.concatenate([x, jnp.zeros((NP - N, 128), _f32)])

    deg = _compute_deg(dstp, ewp).reshape(NP, 1)

    ha, hb = _mlp_stage(xp, deg, W1, b1.reshape(1, -1), W2,
                        b2.reshape(1, -1), Wg1)
    agg1 = _aggregate(ha, hb, srcp, dstp, ewp)
    h2a, h2b = _gcn_mid_stage(agg1, ha, hb, deg, bg1.reshape(1, -1), Wg2)
    agg2 = _aggregate(h2a, h2b, srcp, dstp, ewp)
    out = _final_stage(agg2, h2a, h2b, deg, bg2.reshape(1, -1), Wl,
                       bl.reshape(1, -1), Wgen, bgen.reshape(1, -1))
    return out[:N]
